# baseline (speedup 1.0000x reference)
import functools

import jax
import jax.numpy as jnp
from jax import lax
from jax.experimental import pallas as pl
from jax.experimental.pallas import tpu as pltpu
from jax.experimental.pallas import tpu_sc as plsc

F = 768
S = 2048
E = 8
K = 7
G = 3
TS = 256
NT = S // TS

_HI = lax.Precision.HIGHEST


def _dot(a, b):
    return lax.dot_general(a, b, (((1,), (0,)), ((), ())),
                           preferred_element_type=jnp.float32, precision=_HI)


def _bdot(a, b):
    return lax.dot_general(a.astype(jnp.bfloat16), b.astype(jnp.bfloat16),
                           (((1,), (0,)), ((), ())),
                           preferred_element_type=jnp.float32)


def _router(x, gate):
    logits = lax.dot_general(x.astype(jnp.bfloat16),
                             gate.astype(jnp.bfloat16),
                             (((1,), (1,)), ((), ())),
                             preferred_element_type=jnp.float32)
    m = jnp.max(logits, axis=1, keepdims=True)
    lane = lax.broadcasted_iota(jnp.int32, logits.shape, 1)
    assign = jnp.min(jnp.where(logits == m, lane, E), axis=1, keepdims=True)
    return logits, assign


def _router0_body(x_ref, gate_ref, lg_ref, oh_ref):
    logits, assign = _router(x_ref[...], gate_ref[...])
    lg_ref[...] = logits
    lane = lax.broadcasted_iota(jnp.int32, logits.shape, 1)
    oh_ref[...] = (assign == lane).astype(jnp.bfloat16)


def _router0_call(x, gate):
    return pl.pallas_call(
        _router0_body,
        grid=(NT,),
        in_specs=[pl.BlockSpec((TS, F), lambda t: (t, 0)),
                  pl.BlockSpec((E, F), lambda t: (0, 0))],
        out_specs=[pl.BlockSpec((TS, E), lambda t: (t, 0)),
                   pl.BlockSpec((TS, E), lambda t: (t, 0))],
        out_shape=[jax.ShapeDtypeStruct((S, E), jnp.float32),
                   jax.ShapeDtypeStruct((S, E), jnp.bfloat16)],
    )(x, gate)


def _pos_body(oh_ref, pos_ref, offs_ref):
    t = pl.program_id(0)
    ohf = oh_ref[...]
    ones_row = jnp.ones((1, S), jnp.bfloat16)
    counts = _bdot(ones_row, ohf)
    sl = (lax.broadcasted_iota(jnp.int32, (E, E), 0)
          < lax.broadcasted_iota(jnp.int32, (E, E), 1)).astype(jnp.float32)
    offs_row = _dot(counts, sl)
    offs_ref[...] = offs_row
    oh_tile = oh_ref[pl.ds(t * TS, TS), :].astype(jnp.float32)
    col = lax.broadcasted_iota(jnp.int32, (TS, S), 1)
    row = lax.broadcasted_iota(jnp.int32, (TS, S), 0) + t * TS
    lrow = (col < row).astype(jnp.bfloat16)
    ranks_e = _bdot(lrow, ohf)
    rank = jnp.sum(ranks_e * oh_tile, axis=1, keepdims=True)
    cnt_lt = jnp.sum(offs_row * oh_tile, axis=1, keepdims=True)
    pos_ref[...] = rank + cnt_lt


def _pos_call(oh):
    return pl.pallas_call(
        _pos_body,
        grid=(NT,),
        in_specs=[pl.BlockSpec((S, E), lambda t: (0, 0))],
        out_specs=[pl.BlockSpec((TS, 1), lambda t: (t, 0)),
                   pl.BlockSpec((1, E), lambda t: (0, 0))],
        out_shape=[jax.ShapeDtypeStruct((S, 1), jnp.float32),
                   jax.ShapeDtypeStruct((1, E), jnp.float32)],
    )(oh)


def _ids_body(pos_ref, ids_ref):
    t = pl.program_id(0)
    rowi = (lax.broadcasted_iota(jnp.int32, (TS, S), 0) + t * TS)
    m = (pos_ref[...] == rowi.astype(jnp.float32)).astype(jnp.float32)
    svec = lax.broadcasted_iota(jnp.int32, (S, 1), 0).astype(jnp.float32)
    ids_ref[...] = _dot(m, svec).astype(jnp.int32)


def _ids_call(pos_row):
    return pl.pallas_call(
        _ids_body,
        grid=(NT,),
        in_specs=[pl.BlockSpec((1, S), lambda t: (0, 0))],
        out_specs=pl.BlockSpec((TS, 1), lambda t: (t, 0)),
        out_shape=jax.ShapeDtypeStruct((S, 1), jnp.int32),
    )(pos_row)


def _sc_gather(table, idx):
    n_rows, n_cols = table.shape
    nc, ns = 2, 16
    bpw = n_rows // (nc * ns)
    mesh = plsc.VectorSubcoreMesh(core_axis_name="c", subcore_axis_name="s")

    @functools.partial(
        pl.kernel, mesh=mesh,
        out_type=jax.ShapeDtypeStruct((n_rows, n_cols), table.dtype),
        scratch_types=[
            pltpu.VMEM((bpw,), jnp.int32),
            pltpu.VMEM((bpw, n_cols), table.dtype),
            pltpu.SemaphoreType.DMA,
        ],
    )
    def gather_k(table_hbm, idx_hbm, out_hbm, idx_v, rows_v, sem):
        wid = lax.axis_index("s") * nc + lax.axis_index("c")
        base = wid * bpw
        pltpu.sync_copy(idx_hbm.at[pl.ds(base, bpw)], idx_v)
        pltpu.async_copy(table_hbm.at[idx_v], rows_v, sem).wait()
        pltpu.sync_copy(rows_v, out_hbm.at[pl.ds(base, bpw)])

    return gather_k(table, idx)


def _gmm_body(offs_ref, xs_ref, w_ref, y_ref, *, relu_out):
    t = pl.program_id(1)
    xsb = xs_ref[...].astype(jnp.bfloat16)
    pcol = lax.broadcasted_iota(jnp.int32, (TS, 1), 0) + t * TS
    y_ref[...] = jnp.zeros((TS, F), jnp.float32)
    for e in range(E):
        st = offs_ref[e]
        en = offs_ref[e + 1]

        @pl.when((en > t * TS) & (st < (t + 1) * TS))
        def _():
            pe = lax.dot_general(xsb, w_ref[e], (((1,), (0,)), ((), ())),
                                 preferred_element_type=jnp.float32)
            mask = (pcol >= st) & (pcol < en)
            y_ref[...] += jnp.where(mask, pe, 0.0)
    if relu_out:
        y_ref[...] = jnp.maximum(y_ref[...], 0.0)


def _gmm_call(offs9, xs, wb, oc, relu_out):
    body = functools.partial(_gmm_body, relu_out=relu_out)
    return pl.pallas_call(
        body,
        grid_spec=pltpu.PrefetchScalarGridSpec(
            num_scalar_prefetch=1,
            grid=(oc, NT),
            in_specs=[
                pl.BlockSpec((TS, F), lambda j, t, offs: (t, 0)),
                pl.BlockSpec((E, F, F), lambda j, t, offs: (0, 0, j)),
            ],
            out_specs=pl.BlockSpec((TS, F), lambda j, t, offs: (t, j)),
        ),
        out_shape=jax.ShapeDtypeStruct((S, oc * F), jnp.float32),
    )(offs9, xs, wb)


def _conv_body(x_ref, w_ref, gate_ref, x3_ref, lg_ref, oh_ref):
    t = pl.program_id(0)
    win = x_ref[pl.ds(t * TS, TS + 8), :]
    acc = jnp.zeros((TS, F), jnp.float32)
    for k in range(K):
        acc = acc + _bdot(lax.slice_in_dim(win, k, k + TS, axis=0), w_ref[k])
    x3 = jnp.maximum(acc, 0.0)
    logits, assign = _router(x3, gate_ref[...])
    lg_ref[...] = logits
    lane = lax.broadcasted_iota(jnp.int32, logits.shape, 1)
    oh_ref[...] = (assign == lane).astype(jnp.bfloat16)
    x3_ref[...] = x3


def _conv_call(x, wk, gate):
    xp = jnp.pad(x, ((K - 1, 2), (0, 0)))
    return pl.pallas_call(
        _conv_body,
        grid=(NT,),
        in_specs=[pl.BlockSpec((S + K + 1, F), lambda t: (0, 0)),
                  pl.BlockSpec((K, F, F), lambda t: (0, 0, 0)),
                  pl.BlockSpec((E, F), lambda t: (0, 0))],
        out_specs=[pl.BlockSpec((TS, F), lambda t: (t, 0)),
                   pl.BlockSpec((TS, E), lambda t: (t, 0)),
                   pl.BlockSpec((TS, E), lambda t: (t, 0))],
        out_shape=[jax.ShapeDtypeStruct((S, F), jnp.float32),
                   jax.ShapeDtypeStruct((S, E), jnp.float32),
                   jax.ShapeDtypeStruct((S, E), jnp.bfloat16)],
    )(xp, wk, gate)


def _norm_body(y_ref, u_ref, div_ref, out_ref):
    depth = y_ref[:, 0:F]
    scale = y_ref[:, F:2 * F]
    shift = y_ref[:, 2 * F:3 * F]
    cum = _dot(depth, u_ref[...])
    t = cum / div_ref[...] * scale + shift
    mu = jnp.mean(t, axis=1, keepdims=True)
    c = t - mu
    nrm = jnp.sqrt(jnp.sum(c * c, axis=1, keepdims=True))
    out_ref[...] = c / (nrm * (F ** -0.5) + 1e-5)


def _norm_call(y, u, div_row):
    return pl.pallas_call(
        _norm_body,
        grid=(NT,),
        in_specs=[pl.BlockSpec((TS, G * F), lambda t: (t, 0)),
                  pl.BlockSpec((F, F), lambda t: (0, 0)),
                  pl.BlockSpec((1, F), lambda t: (0, 0))],
        out_specs=pl.BlockSpec((TS, F), lambda t: (t, 0)),
        out_shape=jax.ShapeDtypeStruct((S, F), jnp.float32),
    )(y, u, div_row)


def _loss_body(lg0_ref, lg1_ref, l0_ref, l1_ref):
    for lg_ref, l_ref in ((lg0_ref, l0_ref), (lg1_ref, l1_ref)):
        logits = lg_ref[...]
        m = jnp.max(logits, axis=1, keepdims=True)
        ex = jnp.exp(logits - m)
        p = ex / jnp.sum(ex, axis=1, keepdims=True)
        lane = lax.broadcasted_iota(jnp.int32, logits.shape, 1)
        assign = jnp.min(jnp.where(logits == m, lane, E), axis=1,
                         keepdims=True)
        oh = (assign == lane).astype(jnp.float32)
        gsum = jnp.sum(p, axis=0, keepdims=True)
        csum = jnp.sum(oh, axis=0, keepdims=True)
        l_ref[...] = jnp.sum(gsum * csum, axis=1, keepdims=True) / (S * S)


def _loss_call(lg0, lg1):
    l0, l1 = pl.pallas_call(
        _loss_body,
        in_specs=[pl.BlockSpec((S, E), lambda: (0, 0)),
                  pl.BlockSpec((S, E), lambda: (0, 0))],
        out_specs=[pl.BlockSpec((1, 1), lambda: (0, 0)),
                   pl.BlockSpec((1, 1), lambda: (0, 0))],
        out_shape=[jax.ShapeDtypeStruct((1, 1), jnp.float32),
                   jax.ShapeDtypeStruct((1, 1), jnp.float32)],
    )(lg0, lg1)
    return l0[0, 0], l1[0, 0]


def _route(oh):
    pos, offs = _pos_call(oh)
    ids = _ids_call(pos.reshape(1, S))
    offs9 = jnp.concatenate(
        [offs[0].astype(jnp.int32), jnp.full((8,), S, jnp.int32)])
    return pos, ids.reshape(S), offs9


def kernel(inp, divisor, w0_gate, w0, w1, w2_gate, w2):
    x = inp[0].T
    gate0 = w0_gate[:, :, 0]
    gate1 = w2_gate[:, :, 0]
    w0b = w0.astype(jnp.bfloat16)
    w1k = jnp.transpose(w1, (2, 1, 0))
    w2b = w2.astype(jnp.bfloat16)
    div_row = divisor[0].T
    u = (lax.broadcasted_iota(jnp.int32, (F, F), 0)
         <= lax.broadcasted_iota(jnp.int32, (F, F), 1)).astype(jnp.float32)

    lg0, oh0 = _router0_call(x, gate0)
    pos0, ids0, offs0 = _route(oh0)
    xs0 = _sc_gather(x, ids0)
    y0s = _gmm_call(offs0, xs0, w0b, 1, True)
    y0 = _sc_gather(y0s, pos0.astype(jnp.int32).reshape(S))
    x3, lg1, oh1 = _conv_call(y0, w1k, gate1)
    pos1, ids1, offs1 = _route(oh1)
    xs1 = _sc_gather(x3, ids1)
    y2s = _gmm_call(offs1, xs1, w2b, G, False)
    outs = _norm_call(y2s, u, div_row)
    out = _sc_gather(outs, pos1.astype(jnp.int32).reshape(S))
    l0, l1 = _loss_call(lg0, lg1)
    return (l0, l1, out[None].transpose(0, 2, 1))
